# baseline (speedup 1.0000x reference)
import functools

import jax
import jax.numpy as jnp
import numpy as np
from jax import lax
from jax.experimental import pallas as pl
from jax.experimental.pallas import tpu as pltpu
from jax.experimental.pallas import tpu_sc as plsc

B, F, D, V, DIN = 4096, 26, 64, 1000000, 13
NF = F + 1
NFP = 32
NPAIR = NF * (NF - 1) // 2

_S = np.zeros((NPAIR, NFP * NF), np.float32)
_p = 0
for _n in range(NF):
  for _m in range(_n + 1, NF):
    _S[_p, _n * NFP + _m] = 1.0
    _p += 1

NC, NS = 2, 16
NW = NC * NS
TOTAL = B * F
PER_WB = B // NW
SEG = PER_WB * 128
PER_W = PER_WB * F
CHUNK = 832
NCHUNK = PER_W // CHUNK


def _sc_gather_body(table_hbm, idx_hbm, out_hbm, seg_v, idc_v, rows_v, sem):
  wid = lax.axis_index("s") * NC + lax.axis_index("c")
  pltpu.sync_copy(idx_hbm.at[pl.ds(wid * SEG, SEG)], seg_v)
  lanes = lax.iota(jnp.int32, 16)
  for f in range(F):
    for j in range(PER_WB // 16):
      pos = (j * 16 + lanes) * 128 + f
      idc_v[pl.ds(f * PER_WB + j * 16, 16)] = plsc.load_gather(seg_v, [pos])
  base = wid * PER_W
  for ci in range(NCHUNK):
    idx_sl = idc_v.at[pl.ds(ci * CHUNK, CHUNK)]
    pltpu.async_copy(table_hbm.at[idx_sl], rows_v, sem).wait()
    pltpu.sync_copy(rows_v, out_hbm.at[pl.ds(base + ci * CHUNK, CHUNK)])


def _sc_gather(table, idx_fl):
  mesh = plsc.VectorSubcoreMesh(core_axis_name="c", subcore_axis_name="s")
  fn = functools.partial(
      pl.kernel,
      mesh=mesh,
      out_type=jax.ShapeDtypeStruct((TOTAL, D), jnp.float32),
      scratch_types=[
          pltpu.VMEM((SEG,), jnp.int32),
          pltpu.VMEM((PER_W,), jnp.int32),
          pltpu.VMEM((CHUNK, D), jnp.float32),
          pltpu.SemaphoreType.DMA,
      ],
      compiler_params=pltpu.CompilerParams(
          use_tc_tiling_on_sc=False, needs_layout_passes=False),
  )(_sc_gather_body)
  return fn(table, idx_fl)


BT = 256
GRID = B // BT
H = BT // 4


def _tc_body(xdT_ref, emb_ref, dw1T_ref, db1_ref, dw2T_ref, db2_ref,
             dw3T_ref, db3_ref, ow1aT_ref, w2T_ref, ob1_ref, ow2T_ref,
             ob2_ref, ow3T_ref, ob3_ref, out_ref, c3_ref, g_ref):
  f32 = jnp.float32
  dot = lambda a, b: jax.lax.dot_general(
      a, b, (((1,), (0,)), ((), ())), preferred_element_type=f32)
  h = jnp.maximum(dot(dw1T_ref[...], xdT_ref[...]) + db1_ref[...], 0.0)
  h = jnp.maximum(dot(dw2T_ref[...], h) + db2_ref[...], 0.0)
  doT = jnp.maximum(dot(dw3T_ref[...], h) + db3_ref[...], 0.0)
  d4 = doT.reshape(D, 2, H, 2)
  doTp = jnp.concatenate(
      [d4[:, 0, :, 0], d4[:, 1, :, 0], d4[:, 0, :, 1], d4[:, 1, :, 1]],
      axis=1)
  c3_ref[0] = doTp
  for f in range(F):
    ef = jnp.concatenate([emb_ref[0, f], emb_ref[1, f]], axis=0)
    eT = ef.T
    c3_ref[1 + f] = jnp.concatenate([eT[:D], eT[D:]], axis=1)
  for f in range(NF, NFP):
    c3_ref[f] = jnp.zeros((D, BT), f32)
  c3 = c3_ref[...]
  for n in range(NF):
    prod = c3 * c3[n][None]
    g_ref[pl.ds(n * NFP, NFP)] = jnp.sum(prod, axis=1)
  acc = (dot(w2T_ref[...], g_ref[...]) + dot(ow1aT_ref[...], doTp)
         + ob1_ref[...])
  h = jnp.maximum(acc, 0.0)
  h = jnp.maximum(dot(ow2T_ref[...], h) + ob2_ref[...], 0.0)
  outp = dot(ow3T_ref[...], h) + ob3_ref[...]
  o4 = outp.reshape(8, 2, 2, H)
  nat0 = jnp.stack([o4[:, 0, 0], o4[:, 1, 0]], axis=-1).reshape(8, 2 * H)
  nat1 = jnp.stack([o4[:, 0, 1], o4[:, 1, 1]], axis=-1).reshape(8, 2 * H)
  out_ref[...] = jnp.concatenate([nat0, nat1], axis=1)


def _tc_main(xdT, emb4, dw1T, db1, dw2T, db2, dw3T, db3, ow1aT, w2T, ob1,
             ow2T, ob2, ow3T, ob3, *, interpret=False):
  full = lambda shape: pl.BlockSpec(shape, lambda i: (0,) * len(shape))
  return pl.pallas_call(
      _tc_body,
      grid=(GRID,),
      in_specs=[
          pl.BlockSpec((16, BT), lambda i: (0, i)),
          pl.BlockSpec((2, F, D, 128), lambda i: (i, 0, 0, 0)),
          full((512, 16)), full((512, 1)),
          full((256, 512)), full((256, 1)),
          full((D, 256)), full((D, 1)),
          full((512, D)), full((512, NF * NFP)),
          full((512, 1)), full((256, 512)), full((256, 1)),
          full((8, 256)), full((8, 1)),
      ],
      out_specs=pl.BlockSpec((8, BT), lambda i: (0, i)),
      out_shape=jax.ShapeDtypeStruct((8, B), jnp.float32),
      scratch_shapes=[
          pltpu.VMEM((NFP, D, BT), jnp.float32),
          pltpu.VMEM((NF * NFP, BT), jnp.float32),
      ],
      interpret=interpret,
  )(xdT, emb4, dw1T, db1, dw2T, db2, dw3T, db3, ow1aT, w2T, ob1, ow2T,
    ob2, ow3T, ob3)


def kernel(dense_features, sparse_indices, table, dw1, db1, dw2, db2, dw3,
           db3, ow1, ob1, ow2, ob2, ow3, ob3):
  idxp = lax.optimization_barrier(
      jnp.pad(sparse_indices.astype(jnp.int32), ((0, 0), (0, 128 - F))))
  idx_fl = idxp.reshape(B * 128)
  xdT = jnp.pad(dense_features, ((0, 0), (0, 16 - DIN))).T
  dw1T = jnp.pad(dw1, ((0, 16 - DIN), (0, 0))).T
  ow1aT = ow1[:D].T
  w2T = jnp.dot(ow1[D:].T, jnp.asarray(_S))
  ow3T = jnp.pad(ow3, ((0, 0), (0, 7))).T
  ob3T = jnp.pad(ob3, ((0, 7),)).reshape(8, 1)
  col = lambda b: b.reshape(-1, 1)

  emb = _sc_gather(table, idx_fl)
  emb4 = emb.reshape(NW, F, D, 128)

  out = _tc_main(xdT, emb4, dw1T, col(db1), dw2.T, col(db2), dw3.T,
                 col(db3), ow1aT, w2T, col(ob1), ow2.T, col(ob2), ow3T,
                 ob3T)
  return out[0].reshape(B, 1)
